# baseline (device time: 191612 ns/iter reference)
import jax
import jax.numpy as jnp
from jax import lax
from jax.experimental import pallas as pl
from jax.experimental.pallas import tpu as pltpu

N_DEV = 32
N_HOP = 16
N_FULL = 15
N_SUB = 8
FWD16_SUBS = tuple(range(N_SUB // 2))
BWD16_SUBS = tuple(range(N_SUB // 2, N_SUB))


def _silu(y):
    return y * jax.nn.sigmoid(y)


def _ring_tables():
    plane_order = [(0, 0), (1, 0), (1, 1), (0, 1), (0, 2), (1, 2), (1, 3), (0, 3)]
    coords_by_lid = [(x, y, z) for z in range(4) for (x, y) in plane_order]
    lid = {c: i for i, c in enumerate(coords_by_lid)}

    path_yz = [
        (0, 0), (1, 0), (2, 0), (3, 0), (3, 1), (2, 1), (1, 1), (0, 1),
        (0, 2), (1, 2), (2, 2), (3, 2), (3, 3), (2, 3), (1, 3), (0, 3),
    ]
    cyc = [(0, y, z) for (y, z) in path_yz]
    cyc += [(1, y, z) for (y, z) in reversed(path_yz)]
    sigma = [lid[c] for c in cyc]
    pos = [0] * N_DEV
    for p, l in enumerate(sigma):
        pos[l] = p

    right = [sigma[(pos[l] + 1) % N_DEV] for l in range(N_DEV)]
    left = [sigma[(pos[l] - 1) % N_DEV] for l in range(N_DEV)]
    origf = [
        [sigma[(pos[l] - h) % N_DEV] for l in range(N_DEV)]
        for h in range(N_HOP + 1)
    ]
    origb = [
        [sigma[(pos[l] + h) % N_DEV] for l in range(N_DEV)]
        for h in range(N_HOP + 1)
    ]
    return (
        jnp.array([right, left], dtype=jnp.int32),
        jnp.array(origf, dtype=jnp.int32),
        jnp.array(origb, dtype=jnp.int32),
    )


def kernel(x, w_mat):
    m_per, k = x.shape
    _, n_per = w_mat.shape
    nbr_arr, origf_arr, origb_arr = _ring_tables()

    def body(
        x_ref, w_ref, nbr_ref, origf_ref, origb_ref, out_ref,
        comm0_ref, fwd_ref, bwd_ref, w_bf_ref,
        fwd_send, fwd_recv, bwd_send, bwd_recv,
    ):
        my = lax.axis_index("i")
        right = nbr_ref[0, my]
        left = nbr_ref[1, my]
        sub = m_per // N_SUB

        def fwd_rdma(h, s):
            rows = pl.ds(s * sub, sub)
            src = comm0_ref if h == 1 else fwd_ref.at[h - 1]
            return pltpu.make_async_remote_copy(
                src_ref=src.at[rows],
                dst_ref=fwd_ref.at[h, rows],
                send_sem=fwd_send.at[h - 1, s],
                recv_sem=fwd_recv.at[h - 1, s],
                device_id=(right,),
                device_id_type=pl.DeviceIdType.MESH,
            )

        def bwd_rdma(h, s):
            rows = pl.ds(s * sub, sub)
            src = comm0_ref if h == 1 else bwd_ref.at[h - 1]
            return pltpu.make_async_remote_copy(
                src_ref=src.at[rows],
                dst_ref=bwd_ref.at[h, rows],
                send_sem=bwd_send.at[h - 1, s],
                recv_sem=bwd_recv.at[h - 1, s],
                device_id=(left,),
                device_id_type=pl.DeviceIdType.MESH,
            )

        def gemm_to(origin, chunk):
            out_ref[pl.ds(origin * m_per, m_per), :] = _silu(
                jnp.dot(chunk, w_bf_ref[:, :], preferred_element_type=jnp.float32)
            )

        barrier_sem = pltpu.get_barrier_semaphore()
        for nbr in (left, right):
            pl.semaphore_signal(
                barrier_sem, inc=1,
                device_id=(nbr,), device_id_type=pl.DeviceIdType.MESH,
            )
        pl.semaphore_wait(barrier_sem, 2)

        comm0_ref[:, :] = x_ref[:, :].astype(jnp.bfloat16)

        for s in range(N_SUB):
            fwd_rdma(1, s).start()
            bwd_rdma(1, s).start()
        w_bf_ref[:, :] = w_ref[:, :].astype(jnp.bfloat16)
        gemm_to(my, comm0_ref[:, :])

        for h in range(1, N_FULL + 1):
            for s in range(N_SUB):
                fwd_rdma(h, s).wait_recv()
                if h + 1 <= N_FULL or s in FWD16_SUBS:
                    fwd_rdma(h + 1, s).start()
                bwd_rdma(h, s).wait_recv()
                if h + 1 <= N_FULL or s in BWD16_SUBS:
                    bwd_rdma(h + 1, s).start()
            gemm_to(origf_ref[h, my], fwd_ref[h, :, :])
            gemm_to(origb_ref[h, my], bwd_ref[h, :, :])

        sub_rows = m_per // N_SUB
        anti = origf_ref[N_HOP, my]
        for s in FWD16_SUBS:
            fwd_rdma(N_HOP, s).wait_recv()
            out_ref[pl.ds(anti * m_per + s * sub_rows, sub_rows), :] = _silu(
                jnp.dot(
                    fwd_ref[N_HOP, pl.ds(s * sub_rows, sub_rows), :],
                    w_bf_ref[:, :], preferred_element_type=jnp.float32,
                )
            )
        for s in BWD16_SUBS:
            bwd_rdma(N_HOP, s).wait_recv()
            out_ref[pl.ds(anti * m_per + s * sub_rows, sub_rows), :] = _silu(
                jnp.dot(
                    bwd_ref[N_HOP, pl.ds(s * sub_rows, sub_rows), :],
                    w_bf_ref[:, :], preferred_element_type=jnp.float32,
                )
            )

        for h in range(1, N_FULL + 1):
            for s in range(N_SUB):
                fwd_rdma(h, s).wait_send()
                bwd_rdma(h, s).wait_send()
        for s in FWD16_SUBS:
            fwd_rdma(N_HOP, s).wait_send()
        for s in BWD16_SUBS:
            bwd_rdma(N_HOP, s).wait_send()

    return pl.pallas_call(
        body,
        out_shape=jax.ShapeDtypeStruct((N_DEV * m_per, n_per), jnp.float32),
        in_specs=[
            pl.BlockSpec(memory_space=pltpu.VMEM),
            pl.BlockSpec(memory_space=pltpu.VMEM),
            pl.BlockSpec(memory_space=pltpu.SMEM),
            pl.BlockSpec(memory_space=pltpu.SMEM),
            pl.BlockSpec(memory_space=pltpu.SMEM),
        ],
        out_specs=pl.BlockSpec(memory_space=pltpu.VMEM),
        scratch_shapes=[
            pltpu.VMEM((m_per, k), jnp.bfloat16),
            pltpu.VMEM((N_HOP + 1, m_per, k), jnp.bfloat16),
            pltpu.VMEM((N_HOP + 1, m_per, k), jnp.bfloat16),
            pltpu.VMEM((k, n_per), jnp.bfloat16),
            pltpu.SemaphoreType.DMA((N_HOP, N_SUB)),
            pltpu.SemaphoreType.DMA((N_HOP, N_SUB)),
            pltpu.SemaphoreType.DMA((N_HOP, N_SUB)),
            pltpu.SemaphoreType.DMA((N_HOP, N_SUB)),
        ],
        compiler_params=pltpu.CompilerParams(
            collective_id=0,
            vmem_limit_bytes=48 * 1024 * 1024,
        ),
    )(x, w_mat, nbr_arr, origf_arr, origb_arr)


# device time: 188730 ns/iter; 1.0153x vs baseline; 1.0153x over previous
import jax
import jax.numpy as jnp
from jax import lax
from jax.experimental import pallas as pl
from jax.experimental.pallas import tpu as pltpu

N_DEV = 32
N_HOP = 16
N_FULL = 15
N_SUB = 4
FWD16_SUBS = tuple(range(N_SUB // 2))
BWD16_SUBS = tuple(range(N_SUB // 2, N_SUB))


def _silu(y):
    return y * jax.nn.sigmoid(y)


def _ring_tables():
    plane_order = [(0, 0), (1, 0), (1, 1), (0, 1), (0, 2), (1, 2), (1, 3), (0, 3)]
    coords_by_lid = [(x, y, z) for z in range(4) for (x, y) in plane_order]
    lid = {c: i for i, c in enumerate(coords_by_lid)}

    path_yz = [
        (0, 0), (1, 0), (2, 0), (3, 0), (3, 1), (2, 1), (1, 1), (0, 1),
        (0, 2), (1, 2), (2, 2), (3, 2), (3, 3), (2, 3), (1, 3), (0, 3),
    ]
    cyc = [(0, y, z) for (y, z) in path_yz]
    cyc += [(1, y, z) for (y, z) in reversed(path_yz)]
    sigma = [lid[c] for c in cyc]
    pos = [0] * N_DEV
    for p, l in enumerate(sigma):
        pos[l] = p

    right = [sigma[(pos[l] + 1) % N_DEV] for l in range(N_DEV)]
    left = [sigma[(pos[l] - 1) % N_DEV] for l in range(N_DEV)]
    origf = [
        [sigma[(pos[l] - h) % N_DEV] for l in range(N_DEV)]
        for h in range(N_HOP + 1)
    ]
    origb = [
        [sigma[(pos[l] + h) % N_DEV] for l in range(N_DEV)]
        for h in range(N_HOP + 1)
    ]
    return (
        jnp.array([right, left], dtype=jnp.int32),
        jnp.array(origf, dtype=jnp.int32),
        jnp.array(origb, dtype=jnp.int32),
    )


def kernel(x, w_mat):
    m_per, k = x.shape
    _, n_per = w_mat.shape
    nbr_arr, origf_arr, origb_arr = _ring_tables()

    def body(
        x_ref, w_ref, nbr_ref, origf_ref, origb_ref, out_ref,
        comm0_ref, fwd_ref, bwd_ref, w_bf_ref,
        fwd_send, fwd_recv, bwd_send, bwd_recv,
    ):
        my = lax.axis_index("i")
        right = nbr_ref[0, my]
        left = nbr_ref[1, my]
        sub = m_per // N_SUB

        def fwd_rdma(h, s):
            rows = pl.ds(s * sub, sub)
            src = comm0_ref if h == 1 else fwd_ref.at[h - 1]
            return pltpu.make_async_remote_copy(
                src_ref=src.at[rows],
                dst_ref=fwd_ref.at[h, rows],
                send_sem=fwd_send.at[h - 1, s],
                recv_sem=fwd_recv.at[h - 1, s],
                device_id=(right,),
                device_id_type=pl.DeviceIdType.MESH,
            )

        def bwd_rdma(h, s):
            rows = pl.ds(s * sub, sub)
            src = comm0_ref if h == 1 else bwd_ref.at[h - 1]
            return pltpu.make_async_remote_copy(
                src_ref=src.at[rows],
                dst_ref=bwd_ref.at[h, rows],
                send_sem=bwd_send.at[h - 1, s],
                recv_sem=bwd_recv.at[h - 1, s],
                device_id=(left,),
                device_id_type=pl.DeviceIdType.MESH,
            )

        def gemm_to(origin, chunk):
            out_ref[pl.ds(origin * m_per, m_per), :] = _silu(
                jnp.dot(chunk, w_bf_ref[:, :], preferred_element_type=jnp.float32)
            )

        barrier_sem = pltpu.get_barrier_semaphore()
        for nbr in (left, right):
            pl.semaphore_signal(
                barrier_sem, inc=1,
                device_id=(nbr,), device_id_type=pl.DeviceIdType.MESH,
            )
        pl.semaphore_wait(barrier_sem, 2)

        comm0_ref[:, :] = x_ref[:, :].astype(jnp.bfloat16)

        for s in range(N_SUB):
            fwd_rdma(1, s).start()
            bwd_rdma(1, s).start()
        w_bf_ref[:, :] = w_ref[:, :].astype(jnp.bfloat16)
        gemm_to(my, comm0_ref[:, :])

        for h in range(1, N_FULL + 1):
            for s in range(N_SUB):
                fwd_rdma(h, s).wait_recv()
                if h + 1 <= N_FULL or s in FWD16_SUBS:
                    fwd_rdma(h + 1, s).start()
                bwd_rdma(h, s).wait_recv()
                if h + 1 <= N_FULL or s in BWD16_SUBS:
                    bwd_rdma(h + 1, s).start()
            gemm_to(origf_ref[h, my], fwd_ref[h, :, :])
            gemm_to(origb_ref[h, my], bwd_ref[h, :, :])

        sub_rows = m_per // N_SUB
        anti = origf_ref[N_HOP, my]
        for s in FWD16_SUBS:
            fwd_rdma(N_HOP, s).wait_recv()
            out_ref[pl.ds(anti * m_per + s * sub_rows, sub_rows), :] = _silu(
                jnp.dot(
                    fwd_ref[N_HOP, pl.ds(s * sub_rows, sub_rows), :],
                    w_bf_ref[:, :], preferred_element_type=jnp.float32,
                )
            )
        for s in BWD16_SUBS:
            bwd_rdma(N_HOP, s).wait_recv()
            out_ref[pl.ds(anti * m_per + s * sub_rows, sub_rows), :] = _silu(
                jnp.dot(
                    bwd_ref[N_HOP, pl.ds(s * sub_rows, sub_rows), :],
                    w_bf_ref[:, :], preferred_element_type=jnp.float32,
                )
            )

        for h in range(1, N_FULL + 1):
            for s in range(N_SUB):
                fwd_rdma(h, s).wait_send()
                bwd_rdma(h, s).wait_send()
        for s in FWD16_SUBS:
            fwd_rdma(N_HOP, s).wait_send()
        for s in BWD16_SUBS:
            bwd_rdma(N_HOP, s).wait_send()

    return pl.pallas_call(
        body,
        out_shape=jax.ShapeDtypeStruct((N_DEV * m_per, n_per), jnp.float32),
        in_specs=[
            pl.BlockSpec(memory_space=pltpu.VMEM),
            pl.BlockSpec(memory_space=pltpu.VMEM),
            pl.BlockSpec(memory_space=pltpu.SMEM),
            pl.BlockSpec(memory_space=pltpu.SMEM),
            pl.BlockSpec(memory_space=pltpu.SMEM),
        ],
        out_specs=pl.BlockSpec(memory_space=pltpu.VMEM),
        scratch_shapes=[
            pltpu.VMEM((m_per, k), jnp.bfloat16),
            pltpu.VMEM((N_HOP + 1, m_per, k), jnp.bfloat16),
            pltpu.VMEM((N_HOP + 1, m_per, k), jnp.bfloat16),
            pltpu.VMEM((k, n_per), jnp.bfloat16),
            pltpu.SemaphoreType.DMA((N_HOP, N_SUB)),
            pltpu.SemaphoreType.DMA((N_HOP, N_SUB)),
            pltpu.SemaphoreType.DMA((N_HOP, N_SUB)),
            pltpu.SemaphoreType.DMA((N_HOP, N_SUB)),
        ],
        compiler_params=pltpu.CompilerParams(
            collective_id=0,
            vmem_limit_bytes=48 * 1024 * 1024,
        ),
    )(x, w_mat, nbr_arr, origf_arr, origb_arr)
